# baseline (speedup 1.0000x reference)
import functools
import numpy as np
import jax
import jax.numpy as jnp
from jax import lax
from jax.experimental import pallas as pl
from jax.experimental.pallas import tpu as pltpu
from jax.experimental.pallas import tpu_sc as plsc

D = 128
B = 4
H = 384
NC, NS, L = 2, 16, 16
NW = NC * NS
ROWS_PER_CHUNK = 16
CHUNKS_PER_B = H // ROWS_PER_CHUNK
NTASK = B * CHUNKS_PER_B
CHUNKS_PER_W = NTASK // NW
VPR = H // L

H0, S0, WIN0 = 96, 4, 6
H1, S1, WIN1 = 48, 8, 4
K0 = WIN0 * H0
K1 = WIN1 * H1
SLAB0 = D * K0
SLAB1 = D * K1
DEPW = ROWS_PER_CHUNK * H

STEP = 7.8125
INV_STEP = 0.128


def _sc_stage(depths_flat):
    mesh = plsc.VectorSubcoreMesh(
        core_axis_name="c", subcore_axis_name="s", num_cores=NC, num_subcores=NS
    )

    out_type = (
        jax.ShapeDtypeStruct((NTASK, SLAB0), jnp.float32),
        jax.ShapeDtypeStruct((NTASK, SLAB1), jnp.float32),
    )

    @functools.partial(
        pl.kernel,
        out_type=out_type,
        mesh=mesh,
        compiler_params=pltpu.CompilerParams(needs_layout_passes=False),
        scratch_types=[
            pltpu.VMEM((SLAB0,), jnp.float32),
            pltpu.VMEM((SLAB1,), jnp.float32),
            pltpu.VMEM((DEPW,), jnp.float32),
        ],
    )
    def sc_kernel(dep_hbm, out0_hbm, out1_hbm, slab0, slab1, dep):
        wid = lax.axis_index("s") * NC + lax.axis_index("c")

        zeros16 = jnp.zeros((L,), jnp.float32)
        lane = lax.iota(jnp.int32, L)
        lanef = lane.astype(jnp.float32)
        xi0a = lax.div(2 * lane + (S0 + 1), jnp.full((L,), 2 * S0, jnp.int32)) - 1
        fxa = lanef * (1.0 / S0) + (0.5 / S0 - 0.5) - xi0a.astype(jnp.float32)
        xi1a = xi0a + 1
        wx0a = 1.0 - fxa
        wx1a = fxa
        xi0b = lax.div(2 * lane + (S1 + 1), jnp.full((L,), 2 * S1, jnp.int32)) - 1
        fxb = lanef * (1.0 / S1) + (0.5 / S1 - 0.5) - xi0b.astype(jnp.float32)
        xi1b = xi0b + 1
        wx0b = 1.0 - fxb
        wx1b = fxb

        for c in range(CHUNKS_PER_W):
            t = c * NW + wid
            b = t // CHUNKS_PER_B
            rc = t - b * CHUNKS_PER_B
            base0 = jnp.maximum(S0 * rc - 1, 0)
            base1 = jnp.maximum(2 * rc - 1, 0)

            @plsc.parallel_loop(0, SLAB0 // L, unroll=8)
            def _(i):
                slab0[pl.ds(i * L, L)] = zeros16

            @plsc.parallel_loop(0, SLAB1 // L, unroll=8)
            def _(i):
                slab1[pl.ds(i * L, L)] = zeros16

            pltpu.sync_copy(
                dep_hbm.at[pl.ds(b * (H * H) + rc * DEPW, DEPW)], dep
            )

            def row_body(r, _):
                y = rc * ROWS_PER_CHUNK + r
                yv = jnp.broadcast_to(y, (L,)).astype(jnp.float32)
                y0a = lax.div(2 * y + (S0 + 1), 2 * S0) - 1
                y0av = jnp.broadcast_to(y0a, (L,)).astype(jnp.float32)
                fya = yv * (1.0 / S0) + (0.5 / S0 - 0.5) - y0av
                ly0a = jnp.clip(y0a, 0, H0 - 1) - base0
                ly1a = jnp.clip(y0a + 1, 0, H0 - 1) - base0
                ha = jnp.broadcast_to(ly0a * H0, (L,))
                hb = jnp.broadcast_to(ly1a * H0, (L,))
                w00 = (1.0 - fya) * wx0a
                w01 = (1.0 - fya) * wx1a
                w10 = fya * wx0a
                w11 = fya * wx1a
                y0b = lax.div(2 * y + (S1 + 1), 2 * S1) - 1
                y0bv = jnp.broadcast_to(y0b, (L,)).astype(jnp.float32)
                fyb = yv * (1.0 / S1) + (0.5 / S1 - 0.5) - y0bv
                ly0b = jnp.clip(y0b, 0, H1 - 1) - base1
                ly1b = jnp.clip(y0b + 1, 0, H1 - 1) - base1
                hc = jnp.broadcast_to(ly0b * H1, (L,))
                hd = jnp.broadcast_to(ly1b * H1, (L,))
                v00 = (1.0 - fyb) * wx0b
                v01 = (1.0 - fyb) * wx1b
                v10 = fyb * wx0b
                v11 = fyb * wx1b

                @plsc.parallel_loop(0, VPR, unroll=8)
                def _(v):
                    d = dep[pl.ds((r * VPR + v) * L, L)]
                    q = d * INV_STEP
                    b0 = q.astype(jnp.int32)
                    b1 = jnp.where(b0.astype(jnp.float32) * STEP > d, b0 - 1, b0)
                    b2 = jnp.where(
                        (b1.astype(jnp.float32) + 1.0) * STEP <= d, b1 + 1, b1
                    )
                    bin_ = jnp.clip(b2, 0, D - 1)

                    ta = bin_ * K0
                    xsa = jnp.broadcast_to((L // S0) * v, (L,))
                    x0 = jnp.maximum(xi0a + xsa, 0) + ta
                    x1 = jnp.minimum(xi1a + xsa, H0 - 1) + ta
                    plsc.addupdate_scatter(slab0, [ha + x0], w00)
                    plsc.addupdate_scatter(slab0, [ha + x1], w01)
                    plsc.addupdate_scatter(slab0, [hb + x0], w10)
                    plsc.addupdate_scatter(slab0, [hb + x1], w11)

                    tb = bin_ * K1
                    xsb = jnp.broadcast_to((L // S1) * v, (L,))
                    xb0 = jnp.maximum(xi0b + xsb, 0) + tb
                    xb1 = jnp.minimum(xi1b + xsb, H1 - 1) + tb
                    plsc.addupdate_scatter(slab1, [hc + xb0], v00)
                    plsc.addupdate_scatter(slab1, [hc + xb1], v01)
                    plsc.addupdate_scatter(slab1, [hd + xb0], v10)
                    plsc.addupdate_scatter(slab1, [hd + xb1], v11)

                return 0

            lax.fori_loop(0, ROWS_PER_CHUNK, row_body, 0)

            pltpu.sync_copy(slab0, out0_hbm.at[t])
            pltpu.sync_copy(slab1, out1_hbm.at[t])

    return sc_kernel(depths_flat)


def _tc_stage(slab0, slab1, f0t, f1t):
    C0, C1 = f0t.shape[-1], f1t.shape[-1]
    K = max(C0, C1)

    def body(slab0_ref, slab1_ref, f0_ref, f1_ref, out_ref, acc0, acc1, accc):
        t = pl.program_id(0)

        @pl.when(t == 0)
        def _():
            acc0[...] = jnp.zeros_like(acc0)
            acc1[...] = jnp.zeros_like(acc1)
            accc[...] = jnp.zeros_like(accc)

        b = t // CHUNKS_PER_B
        rc = t - b * CHUNKS_PER_B
        base0 = jnp.maximum(S0 * rc - 1, 0)
        base1 = jnp.maximum(2 * rc - 1, 0)

        s0 = slab0_ref[0]
        win0 = f0_ref[b, pl.ds(base0, WIN0)].reshape(K0, C0)
        win1 = f1_ref[b, pl.ds(base1, WIN1)].reshape(K1, C1)
        acc0[...] += jnp.dot(
            s0, win0,
            preferred_element_type=jnp.float32, precision=lax.Precision.HIGHEST,
        )
        acc1[...] += jnp.dot(
            slab1_ref[0], win1,
            preferred_element_type=jnp.float32, precision=lax.Precision.HIGHEST,
        )
        accc[...] += jnp.sum(s0, axis=1, keepdims=True)

        @pl.when(t == NTASK - 1)
        def _():
            counts = accc[...]
            denom = jnp.maximum(counts, 1.0)
            scale = jnp.where(counts > 0.0, 1.0 / denom, 0.0)
            out_ref[0, : C0, :] = (acc0[...] * scale).T
            out_ref[0, C0:, :] = jnp.zeros((K - C0, D), jnp.float32)
            out_ref[1, :, :] = (acc1[...] * scale).T

    return pl.pallas_call(
        body,
        grid=(NTASK,),
        in_specs=[
            pl.BlockSpec((1, D, K0), lambda t: (t, 0, 0)),
            pl.BlockSpec((1, D, K1), lambda t: (t, 0, 0)),
            pl.BlockSpec((B, H0, H0, C0), lambda t: (0, 0, 0, 0)),
            pl.BlockSpec((B, H1, H1, C1), lambda t: (0, 0, 0, 0)),
        ],
        out_specs=pl.BlockSpec((2, K, D), lambda t: (0, 0, 0)),
        out_shape=jax.ShapeDtypeStruct((2, K, D), jnp.float32),
        scratch_shapes=[
            pltpu.VMEM((D, C0), jnp.float32),
            pltpu.VMEM((D, C1), jnp.float32),
            pltpu.VMEM((D, 1), jnp.float32),
        ],
    )(slab0, slab1, f0t, f1t)


def kernel(imgs, depths, fmap0, fmap1):
    del imgs
    f0t = jnp.transpose(fmap0, (0, 2, 3, 1))
    f1t = jnp.transpose(fmap1, (0, 2, 3, 1))
    depths_flat = depths.reshape(-1)
    slab0, slab1 = _sc_stage(depths_flat)
    slab0 = slab0.reshape(NTASK, D, K0)
    slab1 = slab1.reshape(NTASK, D, K1)
    return _tc_stage(slab0, slab1, f0t, f1t)


# speedup vs baseline: 1.0457x; 1.0457x over previous
import functools
import numpy as np
import jax
import jax.numpy as jnp
from jax import lax
from jax.experimental import pallas as pl
from jax.experimental.pallas import tpu as pltpu
from jax.experimental.pallas import tpu_sc as plsc

D = 128
B = 4
H = 384
NC, NS, L = 2, 16, 16
NW = NC * NS
ROWS_PER_CHUNK = 16
CHUNKS_PER_B = H // ROWS_PER_CHUNK
NTASK = B * CHUNKS_PER_B
CHUNKS_PER_W = NTASK // NW
VPR = H // L

H0, S0, WIN0 = 96, 4, 6
H1, S1, WIN1 = 48, 8, 4
K0 = WIN0 * H0
K1 = WIN1 * H1
SLAB0 = D * K0
SLAB1 = D * K1
DEPW = ROWS_PER_CHUNK * H

STEP = 7.8125
INV_STEP = 0.128


def _sc_stage(depths_flat):
    mesh = plsc.VectorSubcoreMesh(
        core_axis_name="c", subcore_axis_name="s", num_cores=NC, num_subcores=NS
    )

    out_type = (
        jax.ShapeDtypeStruct((NTASK, SLAB0), jnp.float32),
        jax.ShapeDtypeStruct((NTASK, SLAB1), jnp.float32),
    )

    @functools.partial(
        pl.kernel,
        out_type=out_type,
        mesh=mesh,
        compiler_params=pltpu.CompilerParams(needs_layout_passes=False),
        scratch_types=[
            pltpu.VMEM((SLAB0,), jnp.float32),
            pltpu.VMEM((SLAB1,), jnp.float32),
            pltpu.VMEM((DEPW,), jnp.float32),
        ],
    )
    def sc_kernel(dep_hbm, out0_hbm, out1_hbm, slab0, slab1, dep):
        wid = lax.axis_index("s") * NC + lax.axis_index("c")

        zeros16 = jnp.zeros((L,), jnp.float32)
        lane = lax.iota(jnp.int32, L)
        lanef = lane.astype(jnp.float32)
        xi0a = lax.div(2 * lane + (S0 + 1), jnp.full((L,), 2 * S0, jnp.int32)) - 1
        fxa = lanef * (1.0 / S0) + (0.5 / S0 - 0.5) - xi0a.astype(jnp.float32)
        xi1a = xi0a + 1
        wx0a = 1.0 - fxa
        wx1a = fxa
        xi0b = lax.div(2 * lane + (S1 + 1), jnp.full((L,), 2 * S1, jnp.int32)) - 1
        fxb = lanef * (1.0 / S1) + (0.5 / S1 - 0.5) - xi0b.astype(jnp.float32)
        xi1b = xi0b + 1
        wx0b = 1.0 - fxb
        wx1b = fxb

        for c in range(CHUNKS_PER_W):
            t = c * NW + wid
            b = t // CHUNKS_PER_B
            rc = t - b * CHUNKS_PER_B
            base0 = jnp.maximum(S0 * rc - 1, 0)
            base1 = jnp.maximum(2 * rc - 1, 0)

            @plsc.parallel_loop(0, SLAB0 // L, unroll=8)
            def _(i):
                slab0[pl.ds(i * L, L)] = zeros16

            @plsc.parallel_loop(0, SLAB1 // L, unroll=8)
            def _(i):
                slab1[pl.ds(i * L, L)] = zeros16

            pltpu.sync_copy(
                dep_hbm.at[pl.ds(b * (H * H) + rc * DEPW, DEPW)], dep
            )

            def row_body(r, _):
                y = rc * ROWS_PER_CHUNK + r
                yv = jnp.broadcast_to(y, (L,)).astype(jnp.float32)
                y0a = lax.div(2 * y + (S0 + 1), 2 * S0) - 1
                y0av = jnp.broadcast_to(y0a, (L,)).astype(jnp.float32)
                fya = yv * (1.0 / S0) + (0.5 / S0 - 0.5) - y0av
                ly0a = jnp.clip(y0a, 0, H0 - 1) - base0
                ly1a = jnp.clip(y0a + 1, 0, H0 - 1) - base0
                ha = jnp.broadcast_to(ly0a * H0, (L,))
                hb = jnp.broadcast_to(ly1a * H0, (L,))
                w00 = (1.0 - fya) * wx0a
                w01 = (1.0 - fya) * wx1a
                w10 = fya * wx0a
                w11 = fya * wx1a
                y0b = lax.div(2 * y + (S1 + 1), 2 * S1) - 1
                y0bv = jnp.broadcast_to(y0b, (L,)).astype(jnp.float32)
                fyb = yv * (1.0 / S1) + (0.5 / S1 - 0.5) - y0bv
                ly0b = jnp.clip(y0b, 0, H1 - 1) - base1
                ly1b = jnp.clip(y0b + 1, 0, H1 - 1) - base1
                hc = jnp.broadcast_to(ly0b * H1, (L,))
                hd = jnp.broadcast_to(ly1b * H1, (L,))
                v00 = (1.0 - fyb) * wx0b
                v01 = (1.0 - fyb) * wx1b
                v10 = fyb * wx0b
                v11 = fyb * wx1b

                @plsc.parallel_loop(0, VPR, unroll=4)
                def _(v):
                    d = dep[pl.ds((r * VPR + v) * L, L)]
                    q = d * INV_STEP
                    b0 = q.astype(jnp.int32)
                    b1 = jnp.where(b0.astype(jnp.float32) * STEP > d, b0 - 1, b0)
                    b2 = jnp.where(
                        (b1.astype(jnp.float32) + 1.0) * STEP <= d, b1 + 1, b1
                    )
                    bin_ = jnp.clip(b2, 0, D - 1)

                    ta = bin_ * K0
                    xsa = jnp.broadcast_to((L // S0) * v, (L,))
                    x0 = jnp.maximum(xi0a + xsa, 0) + ta
                    x1 = jnp.minimum(xi1a + xsa, H0 - 1) + ta
                    plsc.addupdate_scatter(slab0, [ha + x0], w00)
                    plsc.addupdate_scatter(slab0, [ha + x1], w01)
                    plsc.addupdate_scatter(slab0, [hb + x0], w10)
                    plsc.addupdate_scatter(slab0, [hb + x1], w11)

                    tb = bin_ * K1
                    xsb = jnp.broadcast_to((L // S1) * v, (L,))
                    xb0 = jnp.maximum(xi0b + xsb, 0) + tb
                    xb1 = jnp.minimum(xi1b + xsb, H1 - 1) + tb
                    plsc.addupdate_scatter(slab1, [hc + xb0], v00)
                    plsc.addupdate_scatter(slab1, [hc + xb1], v01)
                    plsc.addupdate_scatter(slab1, [hd + xb0], v10)
                    plsc.addupdate_scatter(slab1, [hd + xb1], v11)

                return 0

            lax.fori_loop(0, ROWS_PER_CHUNK, row_body, 0)

            pltpu.sync_copy(slab0, out0_hbm.at[t])
            pltpu.sync_copy(slab1, out1_hbm.at[t])

    return sc_kernel(depths_flat)


def _tc_stage(slab0, slab1, f0t, f1t):
    C0, C1 = f0t.shape[-1], f1t.shape[-1]
    K = max(C0, C1)

    def body(slab0_ref, slab1_ref, f0_ref, f1_ref, out_ref, acc0, acc1, accc):
        t = pl.program_id(0)

        @pl.when(t == 0)
        def _():
            acc0[...] = jnp.zeros_like(acc0)
            acc1[...] = jnp.zeros_like(acc1)
            accc[...] = jnp.zeros_like(accc)

        b = t // CHUNKS_PER_B
        rc = t - b * CHUNKS_PER_B
        base0 = jnp.maximum(S0 * rc - 1, 0)
        base1 = jnp.maximum(2 * rc - 1, 0)

        s0 = slab0_ref[0]
        win0 = f0_ref[b, pl.ds(base0, WIN0)].reshape(K0, C0)
        win1 = f1_ref[b, pl.ds(base1, WIN1)].reshape(K1, C1)
        acc0[...] += jnp.dot(
            s0, win0,
            preferred_element_type=jnp.float32, precision=lax.Precision.HIGHEST,
        )
        acc1[...] += jnp.dot(
            slab1_ref[0], win1,
            preferred_element_type=jnp.float32, precision=lax.Precision.HIGHEST,
        )
        accc[...] += jnp.sum(s0, axis=1, keepdims=True)

        @pl.when(t == NTASK - 1)
        def _():
            counts = accc[...]
            denom = jnp.maximum(counts, 1.0)
            scale = jnp.where(counts > 0.0, 1.0 / denom, 0.0)
            out_ref[0, : C0, :] = (acc0[...] * scale).T
            out_ref[0, C0:, :] = jnp.zeros((K - C0, D), jnp.float32)
            out_ref[1, :, :] = (acc1[...] * scale).T

    return pl.pallas_call(
        body,
        grid=(NTASK,),
        in_specs=[
            pl.BlockSpec((1, D, K0), lambda t: (t, 0, 0)),
            pl.BlockSpec((1, D, K1), lambda t: (t, 0, 0)),
            pl.BlockSpec((B, H0, H0, C0), lambda t: (0, 0, 0, 0)),
            pl.BlockSpec((B, H1, H1, C1), lambda t: (0, 0, 0, 0)),
        ],
        out_specs=pl.BlockSpec((2, K, D), lambda t: (0, 0, 0)),
        out_shape=jax.ShapeDtypeStruct((2, K, D), jnp.float32),
        scratch_shapes=[
            pltpu.VMEM((D, C0), jnp.float32),
            pltpu.VMEM((D, C1), jnp.float32),
            pltpu.VMEM((D, 1), jnp.float32),
        ],
    )(slab0, slab1, f0t, f1t)


def kernel(imgs, depths, fmap0, fmap1):
    del imgs
    f0t = jnp.transpose(fmap0, (0, 2, 3, 1))
    f1t = jnp.transpose(fmap1, (0, 2, 3, 1))
    depths_flat = depths.reshape(-1)
    slab0, slab1 = _sc_stage(depths_flat)
    slab0 = slab0.reshape(NTASK, D, K0)
    slab1 = slab1.reshape(NTASK, D, K1)
    return _tc_stage(slab0, slab1, f0t, f1t)


# speedup vs baseline: 1.1975x; 1.1451x over previous
import functools
import numpy as np
import jax
import jax.numpy as jnp
from jax import lax
from jax.experimental import pallas as pl
from jax.experimental.pallas import tpu as pltpu
from jax.experimental.pallas import tpu_sc as plsc

D = 128
B = 4
H = 384
NC, NS, L = 2, 16, 16
NW = NC * NS
ROWS_PER_CHUNK = 16
CHUNKS_PER_B = H // ROWS_PER_CHUNK
NTASK = B * CHUNKS_PER_B
CHUNKS_PER_W = NTASK // NW
VPR = H // L

H0, S0, WIN0 = 96, 4, 6
H1, S1, WIN1 = 48, 8, 4
K0 = WIN0 * H0
K1 = WIN1 * H1
SLAB0 = D * K0
SLAB1 = D * K1
DEPW = ROWS_PER_CHUNK * H

STEP = 7.8125
INV_STEP = 0.128
TC_SUB = 2


def _sc_stage(depths_flat):
    mesh = plsc.VectorSubcoreMesh(
        core_axis_name="c", subcore_axis_name="s", num_cores=NC, num_subcores=NS
    )

    out_type = (
        jax.ShapeDtypeStruct((NTASK, SLAB0), jnp.float32),
        jax.ShapeDtypeStruct((NTASK, SLAB1), jnp.float32),
    )

    @functools.partial(
        pl.kernel,
        out_type=out_type,
        mesh=mesh,
        compiler_params=pltpu.CompilerParams(needs_layout_passes=False),
        scratch_types=[
            pltpu.VMEM((SLAB0,), jnp.float32),
            pltpu.VMEM((SLAB1,), jnp.float32),
            pltpu.VMEM((2, DEPW), jnp.float32),
            pltpu.SemaphoreType.DMA,
            pltpu.SemaphoreType.DMA,
            pltpu.SemaphoreType.DMA,
        ],
    )
    def sc_kernel(dep_hbm, out0_hbm, out1_hbm, slab0, slab1, dep2, sem0, sem1, semd):
        wid = lax.axis_index("s") * NC + lax.axis_index("c")

        zeros16 = jnp.zeros((L,), jnp.float32)
        lane = lax.iota(jnp.int32, L)
        lanef = lane.astype(jnp.float32)
        xi0a = lax.div(2 * lane + (S0 + 1), jnp.full((L,), 2 * S0, jnp.int32)) - 1
        fxa = lanef * (1.0 / S0) + (0.5 / S0 - 0.5) - xi0a.astype(jnp.float32)
        xi1a = xi0a + 1
        wx0a = 1.0 - fxa
        wx1a = fxa
        xi0b = lax.div(2 * lane + (S1 + 1), jnp.full((L,), 2 * S1, jnp.int32)) - 1
        fxb = lanef * (1.0 / S1) + (0.5 / S1 - 0.5) - xi0b.astype(jnp.float32)
        xi1b = xi0b + 1
        wx0b = 1.0 - fxb
        wx1b = fxb

        def dep_start(c):
            t = c * NW + wid
            b = t // CHUNKS_PER_B
            rc = t - b * CHUNKS_PER_B
            return pltpu.make_async_copy(
                dep_hbm.at[pl.ds(b * (H * H) + rc * DEPW, DEPW)],
                dep2.at[c % 2],
                semd,
            )

        dep_start(0).start()

        for c in range(CHUNKS_PER_W):
            t = c * NW + wid
            b = t // CHUNKS_PER_B
            rc = t - b * CHUNKS_PER_B
            base0 = jnp.maximum(S0 * rc - 1, 0)
            base1 = jnp.maximum(2 * rc - 1, 0)
            cb = c % 2

            if c == 0:
                @plsc.parallel_loop(0, SLAB0 // L, unroll=8)
                def _(i):
                    slab0[pl.ds(i * L, L)] = zeros16

                @plsc.parallel_loop(0, SLAB1 // L, unroll=8)
                def _(i):
                    slab1[pl.ds(i * L, L)] = zeros16

            dep_start(c).wait()
            if c + 1 < CHUNKS_PER_W:
                dep_start(c + 1).start()

            def row_body(r, _):
                y = rc * ROWS_PER_CHUNK + r
                yv = jnp.broadcast_to(y, (L,)).astype(jnp.float32)
                y0a = lax.div(2 * y + (S0 + 1), 2 * S0) - 1
                y0av = jnp.broadcast_to(y0a, (L,)).astype(jnp.float32)
                fya = yv * (1.0 / S0) + (0.5 / S0 - 0.5) - y0av
                ly0a = jnp.clip(y0a, 0, H0 - 1) - base0
                ly1a = jnp.clip(y0a + 1, 0, H0 - 1) - base0
                ha = jnp.broadcast_to(ly0a * H0, (L,))
                hb = jnp.broadcast_to(ly1a * H0, (L,))
                w00 = (1.0 - fya) * wx0a
                w01 = (1.0 - fya) * wx1a
                w10 = fya * wx0a
                w11 = fya * wx1a
                y0b = lax.div(2 * y + (S1 + 1), 2 * S1) - 1
                y0bv = jnp.broadcast_to(y0b, (L,)).astype(jnp.float32)
                fyb = yv * (1.0 / S1) + (0.5 / S1 - 0.5) - y0bv
                ly0b = jnp.clip(y0b, 0, H1 - 1) - base1
                ly1b = jnp.clip(y0b + 1, 0, H1 - 1) - base1
                hc = jnp.broadcast_to(ly0b * H1, (L,))
                hd = jnp.broadcast_to(ly1b * H1, (L,))
                v00 = (1.0 - fyb) * wx0b
                v01 = (1.0 - fyb) * wx1b
                v10 = fyb * wx0b
                v11 = fyb * wx1b

                @plsc.parallel_loop(0, VPR, unroll=4)
                def _(v):
                    d = dep2[cb, pl.ds((r * VPR + v) * L, L)]
                    q = d * INV_STEP
                    b0 = q.astype(jnp.int32)
                    b1 = jnp.where(b0.astype(jnp.float32) * STEP > d, b0 - 1, b0)
                    b2 = jnp.where(
                        (b1.astype(jnp.float32) + 1.0) * STEP <= d, b1 + 1, b1
                    )
                    bin_ = jnp.clip(b2, 0, D - 1)

                    ta = bin_ * K0
                    xsa = jnp.broadcast_to((L // S0) * v, (L,))
                    x0 = jnp.maximum(xi0a + xsa, 0) + ta
                    x1 = jnp.minimum(xi1a + xsa, H0 - 1) + ta
                    plsc.addupdate_scatter(slab0, [ha + x0], w00)
                    plsc.addupdate_scatter(slab0, [ha + x1], w01)
                    plsc.addupdate_scatter(slab0, [hb + x0], w10)
                    plsc.addupdate_scatter(slab0, [hb + x1], w11)

                    tb = bin_ * K1
                    xsb = jnp.broadcast_to((L // S1) * v, (L,))
                    xb0 = jnp.maximum(xi0b + xsb, 0) + tb
                    xb1 = jnp.minimum(xi1b + xsb, H1 - 1) + tb
                    plsc.addupdate_scatter(slab1, [hc + xb0], v00)
                    plsc.addupdate_scatter(slab1, [hc + xb1], v01)
                    plsc.addupdate_scatter(slab1, [hd + xb0], v10)
                    plsc.addupdate_scatter(slab1, [hd + xb1], v11)

                return 0

            lax.fori_loop(0, ROWS_PER_CHUNK, row_body, 0)

            h0 = pltpu.make_async_copy(slab0, out0_hbm.at[t], sem0)
            h1 = pltpu.make_async_copy(slab1, out1_hbm.at[t], sem1)
            h0.start()
            h1.start()
            h0.wait()
            if c + 1 < CHUNKS_PER_W:
                @plsc.parallel_loop(0, SLAB0 // L, unroll=8)
                def _(i):
                    slab0[pl.ds(i * L, L)] = zeros16

            h1.wait()
            if c + 1 < CHUNKS_PER_W:
                @plsc.parallel_loop(0, SLAB1 // L, unroll=8)
                def _(i):
                    slab1[pl.ds(i * L, L)] = zeros16

    return sc_kernel(depths_flat)


def _tc_stage(slab0, slab1, f0t, f1t):
    C0, C1 = f0t.shape[-1], f1t.shape[-1]
    K = max(C0, C1)

    def body(slab0_ref, slab1_ref, f0_ref, f1_ref, out_ref, acc0, acc1, accc):
        t = pl.program_id(0)

        @pl.when(t == 0)
        def _():
            acc0[...] = jnp.zeros_like(acc0)
            acc1[...] = jnp.zeros_like(acc1)
            accc[...] = jnp.zeros_like(accc)

        for sub in range(TC_SUB):
            tt = TC_SUB * t + sub
            b = tt // CHUNKS_PER_B
            rc = tt - b * CHUNKS_PER_B
            base0 = jnp.maximum(S0 * rc - 1, 0)
            base1 = jnp.maximum(2 * rc - 1, 0)

            s0 = slab0_ref[sub]
            win0 = f0_ref[b, pl.ds(base0, WIN0)].reshape(K0, C0)
            win1 = f1_ref[b, pl.ds(base1, WIN1)].reshape(K1, C1)
            acc0[...] += jnp.dot(
                s0, win0,
                preferred_element_type=jnp.float32,
                precision=lax.Precision.HIGHEST,
            )
            acc1[...] += jnp.dot(
                slab1_ref[sub], win1,
                preferred_element_type=jnp.float32,
                precision=lax.Precision.HIGHEST,
            )
            accc[...] += jnp.sum(s0, axis=1, keepdims=True)

        @pl.when(t == NTASK // TC_SUB - 1)
        def _():
            counts = accc[...]
            denom = jnp.maximum(counts, 1.0)
            scale = jnp.where(counts > 0.0, 1.0 / denom, 0.0)
            out_ref[0, : C0, :] = (acc0[...] * scale).T
            out_ref[0, C0:, :] = jnp.zeros((K - C0, D), jnp.float32)
            out_ref[1, :, :] = (acc1[...] * scale).T

    return pl.pallas_call(
        body,
        grid=(NTASK // TC_SUB,),
        in_specs=[
            pl.BlockSpec((TC_SUB, D, K0), lambda t: (t, 0, 0)),
            pl.BlockSpec((TC_SUB, D, K1), lambda t: (t, 0, 0)),
            pl.BlockSpec((B, H0, H0, C0), lambda t: (0, 0, 0, 0)),
            pl.BlockSpec((B, H1, H1, C1), lambda t: (0, 0, 0, 0)),
        ],
        out_specs=pl.BlockSpec((2, K, D), lambda t: (0, 0, 0)),
        out_shape=jax.ShapeDtypeStruct((2, K, D), jnp.float32),
        scratch_shapes=[
            pltpu.VMEM((D, C0), jnp.float32),
            pltpu.VMEM((D, C1), jnp.float32),
            pltpu.VMEM((D, 1), jnp.float32),
        ],
    )(slab0, slab1, f0t, f1t)


def kernel(imgs, depths, fmap0, fmap1):
    del imgs
    f0t = jnp.transpose(fmap0, (0, 2, 3, 1))
    f1t = jnp.transpose(fmap1, (0, 2, 3, 1))
    depths_flat = depths.reshape(-1)
    slab0, slab1 = _sc_stage(depths_flat)
    slab0 = slab0.reshape(NTASK, D, K0)
    slab1 = slab1.reshape(NTASK, D, K1)
    return _tc_stage(slab0, slab1, f0t, f1t)


# speedup vs baseline: 1.2504x; 1.0442x over previous
import functools
import numpy as np
import jax
import jax.numpy as jnp
from jax import lax
from jax.experimental import pallas as pl
from jax.experimental.pallas import tpu as pltpu
from jax.experimental.pallas import tpu_sc as plsc

D = 128
B = 4
H = 384
NC, NS, L = 2, 16, 16
NW = NC * NS
ROWS_PER_CHUNK = 16
CHUNKS_PER_B = H // ROWS_PER_CHUNK
NTASK = B * CHUNKS_PER_B
CHUNKS_PER_W = NTASK // NW
VPR = H // L

H0, S0, WIN0 = 96, 4, 6
H1, S1, WIN1 = 48, 8, 4
K0 = WIN0 * H0
K1 = WIN1 * H1
SLAB0 = D * K0
SLAB1 = D * K1
DEPW = ROWS_PER_CHUNK * H

STEP = 7.8125
INV_STEP = 0.128
TC_SUB = 4


def _sc_stage(depths_flat):
    mesh = plsc.VectorSubcoreMesh(
        core_axis_name="c", subcore_axis_name="s", num_cores=NC, num_subcores=NS
    )

    out_type = (
        jax.ShapeDtypeStruct((NTASK, SLAB0), jnp.float32),
        jax.ShapeDtypeStruct((NTASK, SLAB1), jnp.float32),
    )

    @functools.partial(
        pl.kernel,
        out_type=out_type,
        mesh=mesh,
        compiler_params=pltpu.CompilerParams(needs_layout_passes=False),
        scratch_types=[
            pltpu.VMEM((SLAB0,), jnp.float32),
            pltpu.VMEM((SLAB1,), jnp.float32),
            pltpu.VMEM((2, DEPW), jnp.float32),
            pltpu.SemaphoreType.DMA,
            pltpu.SemaphoreType.DMA,
            pltpu.SemaphoreType.DMA,
        ],
    )
    def sc_kernel(dep_hbm, out0_hbm, out1_hbm, slab0, slab1, dep2, sem0, sem1, semd):
        wid = lax.axis_index("s") * NC + lax.axis_index("c")

        zeros16 = jnp.zeros((L,), jnp.float32)
        lane = lax.iota(jnp.int32, L)
        lanef = lane.astype(jnp.float32)
        xi0a = lax.div(2 * lane + (S0 + 1), jnp.full((L,), 2 * S0, jnp.int32)) - 1
        fxa = lanef * (1.0 / S0) + (0.5 / S0 - 0.5) - xi0a.astype(jnp.float32)
        xi1a = xi0a + 1
        wx0a = 1.0 - fxa
        wx1a = fxa
        xi0b = lax.div(2 * lane + (S1 + 1), jnp.full((L,), 2 * S1, jnp.int32)) - 1
        fxb = lanef * (1.0 / S1) + (0.5 / S1 - 0.5) - xi0b.astype(jnp.float32)
        xi1b = xi0b + 1
        wx0b = 1.0 - fxb
        wx1b = fxb

        def dep_start(c):
            t = c * NW + wid
            b = t // CHUNKS_PER_B
            rc = t - b * CHUNKS_PER_B
            return pltpu.make_async_copy(
                dep_hbm.at[pl.ds(b * (H * H) + rc * DEPW, DEPW)],
                dep2.at[c % 2],
                semd,
            )

        dep_start(0).start()

        for c in range(CHUNKS_PER_W):
            t = c * NW + wid
            b = t // CHUNKS_PER_B
            rc = t - b * CHUNKS_PER_B
            base0 = jnp.maximum(S0 * rc - 1, 0)
            base1 = jnp.maximum(2 * rc - 1, 0)
            cb = c % 2

            if c == 0:
                @plsc.parallel_loop(0, SLAB0 // L, unroll=8)
                def _(i):
                    slab0[pl.ds(i * L, L)] = zeros16

                @plsc.parallel_loop(0, SLAB1 // L, unroll=8)
                def _(i):
                    slab1[pl.ds(i * L, L)] = zeros16

            dep_start(c).wait()
            if c + 1 < CHUNKS_PER_W:
                dep_start(c + 1).start()

            @plsc.parallel_loop(0, ROWS_PER_CHUNK, unroll=1)
            def row_body(r):
                y = rc * ROWS_PER_CHUNK + r
                yv = jnp.broadcast_to(y, (L,)).astype(jnp.float32)
                y0a = lax.div(2 * y + (S0 + 1), 2 * S0) - 1
                y0av = jnp.broadcast_to(y0a, (L,)).astype(jnp.float32)
                fya = yv * (1.0 / S0) + (0.5 / S0 - 0.5) - y0av
                ly0a = jnp.clip(y0a, 0, H0 - 1) - base0
                ly1a = jnp.clip(y0a + 1, 0, H0 - 1) - base0
                ha = jnp.broadcast_to(ly0a * H0, (L,))
                hb = jnp.broadcast_to(ly1a * H0, (L,))
                w00 = (1.0 - fya) * wx0a
                w01 = (1.0 - fya) * wx1a
                w10 = fya * wx0a
                w11 = fya * wx1a
                y0b = lax.div(2 * y + (S1 + 1), 2 * S1) - 1
                y0bv = jnp.broadcast_to(y0b, (L,)).astype(jnp.float32)
                fyb = yv * (1.0 / S1) + (0.5 / S1 - 0.5) - y0bv
                ly0b = jnp.clip(y0b, 0, H1 - 1) - base1
                ly1b = jnp.clip(y0b + 1, 0, H1 - 1) - base1
                hc = jnp.broadcast_to(ly0b * H1, (L,))
                hd = jnp.broadcast_to(ly1b * H1, (L,))
                v00 = (1.0 - fyb) * wx0b
                v01 = (1.0 - fyb) * wx1b
                v10 = fyb * wx0b
                v11 = fyb * wx1b

                @plsc.parallel_loop(0, VPR, unroll=4)
                def _(v):
                    d = dep2[cb, pl.ds((r * VPR + v) * L, L)]
                    q = d * INV_STEP
                    b0 = q.astype(jnp.int32)
                    b1 = jnp.where(b0.astype(jnp.float32) * STEP > d, b0 - 1, b0)
                    b2 = jnp.where(
                        (b1.astype(jnp.float32) + 1.0) * STEP <= d, b1 + 1, b1
                    )
                    bin_ = jnp.clip(b2, 0, D - 1)

                    ta = bin_ * K0
                    xsa = jnp.broadcast_to((L // S0) * v, (L,))
                    x0 = jnp.maximum(xi0a + xsa, 0) + ta
                    x1 = jnp.minimum(xi1a + xsa, H0 - 1) + ta
                    plsc.addupdate_scatter(slab0, [ha + x0], w00)
                    plsc.addupdate_scatter(slab0, [ha + x1], w01)
                    plsc.addupdate_scatter(slab0, [hb + x0], w10)
                    plsc.addupdate_scatter(slab0, [hb + x1], w11)

                    tb = bin_ * K1
                    xsb = jnp.broadcast_to((L // S1) * v, (L,))
                    xb0 = jnp.maximum(xi0b + xsb, 0) + tb
                    xb1 = jnp.minimum(xi1b + xsb, H1 - 1) + tb
                    plsc.addupdate_scatter(slab1, [hc + xb0], v00)
                    plsc.addupdate_scatter(slab1, [hc + xb1], v01)
                    plsc.addupdate_scatter(slab1, [hd + xb0], v10)
                    plsc.addupdate_scatter(slab1, [hd + xb1], v11)

            h0 = pltpu.make_async_copy(slab0, out0_hbm.at[t], sem0)
            h1 = pltpu.make_async_copy(slab1, out1_hbm.at[t], sem1)
            h0.start()
            h1.start()
            h0.wait()
            if c + 1 < CHUNKS_PER_W:
                @plsc.parallel_loop(0, SLAB0 // L, unroll=8)
                def _(i):
                    slab0[pl.ds(i * L, L)] = zeros16

            h1.wait()
            if c + 1 < CHUNKS_PER_W:
                @plsc.parallel_loop(0, SLAB1 // L, unroll=8)
                def _(i):
                    slab1[pl.ds(i * L, L)] = zeros16

    return sc_kernel(depths_flat)


def _tc_stage(slab0, slab1, f0t, f1t):
    C0, C1 = f0t.shape[-1], f1t.shape[-1]
    K = max(C0, C1)

    def body(slab0_ref, slab1_ref, f0_ref, f1_ref, out_ref, acc0, acc1, accc):
        t = pl.program_id(0)

        @pl.when(t == 0)
        def _():
            acc0[...] = jnp.zeros_like(acc0)
            acc1[...] = jnp.zeros_like(acc1)
            accc[...] = jnp.zeros_like(accc)

        for sub in range(TC_SUB):
            tt = TC_SUB * t + sub
            b = tt // CHUNKS_PER_B
            rc = tt - b * CHUNKS_PER_B
            base0 = jnp.maximum(S0 * rc - 1, 0)
            base1 = jnp.maximum(2 * rc - 1, 0)

            s0 = slab0_ref[sub]
            win0 = f0_ref[b, pl.ds(base0, WIN0)].reshape(K0, C0)
            win1 = f1_ref[b, pl.ds(base1, WIN1)].reshape(K1, C1)
            acc0[...] += jnp.dot(
                s0, win0,
                preferred_element_type=jnp.float32,
                precision=lax.Precision.HIGHEST,
            )
            acc1[...] += jnp.dot(
                slab1_ref[sub], win1,
                preferred_element_type=jnp.float32,
                precision=lax.Precision.HIGHEST,
            )
            accc[...] += jnp.sum(s0, axis=1, keepdims=True)

        @pl.when(t == NTASK // TC_SUB - 1)
        def _():
            counts = accc[...]
            denom = jnp.maximum(counts, 1.0)
            scale = jnp.where(counts > 0.0, 1.0 / denom, 0.0)
            out_ref[0, : C0, :] = (acc0[...] * scale).T
            out_ref[0, C0:, :] = jnp.zeros((K - C0, D), jnp.float32)
            out_ref[1, :, :] = (acc1[...] * scale).T

    return pl.pallas_call(
        body,
        grid=(NTASK // TC_SUB,),
        in_specs=[
            pl.BlockSpec((TC_SUB, D, K0), lambda t: (t, 0, 0)),
            pl.BlockSpec((TC_SUB, D, K1), lambda t: (t, 0, 0)),
            pl.BlockSpec((B, H0, H0, C0), lambda t: (0, 0, 0, 0)),
            pl.BlockSpec((B, H1, H1, C1), lambda t: (0, 0, 0, 0)),
        ],
        out_specs=pl.BlockSpec((2, K, D), lambda t: (0, 0, 0)),
        out_shape=jax.ShapeDtypeStruct((2, K, D), jnp.float32),
        scratch_shapes=[
            pltpu.VMEM((D, C0), jnp.float32),
            pltpu.VMEM((D, C1), jnp.float32),
            pltpu.VMEM((D, 1), jnp.float32),
        ],
    )(slab0, slab1, f0t, f1t)


def kernel(imgs, depths, fmap0, fmap1):
    del imgs
    f0t = jnp.transpose(fmap0, (0, 2, 3, 1))
    f1t = jnp.transpose(fmap1, (0, 2, 3, 1))
    depths_flat = depths.reshape(-1)
    slab0, slab1 = _sc_stage(depths_flat)
    slab0 = slab0.reshape(NTASK, D, K0)
    slab1 = slab1.reshape(NTASK, D, K1)
    return _tc_stage(slab0, slab1, f0t, f1t)


# speedup vs baseline: 1.2569x; 1.0052x over previous
import functools
import numpy as np
import jax
import jax.numpy as jnp
from jax import lax
from jax.experimental import pallas as pl
from jax.experimental.pallas import tpu as pltpu
from jax.experimental.pallas import tpu_sc as plsc

D = 128
B = 4
H = 384
NC, NS, L = 2, 16, 16
NW = NC * NS
ROWS_PER_CHUNK = 16
CHUNKS_PER_B = H // ROWS_PER_CHUNK
NTASK = B * CHUNKS_PER_B
CHUNKS_PER_W = NTASK // NW
VPR = H // L

H0, S0, WIN0 = 96, 4, 6
H1, S1, WIN1 = 48, 8, 4
K0 = WIN0 * H0
K1 = WIN1 * H1
SLAB0 = D * K0
SLAB1 = D * K1
DEPW = ROWS_PER_CHUNK * H

STEP = 7.8125
INV_STEP = 0.128
TC_SUB = 8


def _sc_stage(depths_flat):
    mesh = plsc.VectorSubcoreMesh(
        core_axis_name="c", subcore_axis_name="s", num_cores=NC, num_subcores=NS
    )

    out_type = (
        jax.ShapeDtypeStruct((NTASK, SLAB0), jnp.float32),
        jax.ShapeDtypeStruct((NTASK, SLAB1), jnp.float32),
    )

    @functools.partial(
        pl.kernel,
        out_type=out_type,
        mesh=mesh,
        compiler_params=pltpu.CompilerParams(needs_layout_passes=False),
        scratch_types=[
            pltpu.VMEM((SLAB0,), jnp.float32),
            pltpu.VMEM((SLAB1,), jnp.float32),
            pltpu.VMEM((2, DEPW), jnp.float32),
            pltpu.SemaphoreType.DMA,
            pltpu.SemaphoreType.DMA,
            pltpu.SemaphoreType.DMA,
        ],
    )
    def sc_kernel(dep_hbm, out0_hbm, out1_hbm, slab0, slab1, dep2, sem0, sem1, semd):
        wid = lax.axis_index("s") * NC + lax.axis_index("c")

        zeros16 = jnp.zeros((L,), jnp.float32)
        lane = lax.iota(jnp.int32, L)
        lanef = lane.astype(jnp.float32)
        xi0a = lax.div(2 * lane + (S0 + 1), jnp.full((L,), 2 * S0, jnp.int32)) - 1
        fxa = lanef * (1.0 / S0) + (0.5 / S0 - 0.5) - xi0a.astype(jnp.float32)
        xi1a = xi0a + 1
        wx0a = 1.0 - fxa
        wx1a = fxa
        xi0b = lax.div(2 * lane + (S1 + 1), jnp.full((L,), 2 * S1, jnp.int32)) - 1
        fxb = lanef * (1.0 / S1) + (0.5 / S1 - 0.5) - xi0b.astype(jnp.float32)
        xi1b = xi0b + 1
        wx0b = 1.0 - fxb
        wx1b = fxb

        def dep_start(c):
            t = c * NW + wid
            b = t // CHUNKS_PER_B
            rc = t - b * CHUNKS_PER_B
            return pltpu.make_async_copy(
                dep_hbm.at[pl.ds(b * (H * H) + rc * DEPW, DEPW)],
                dep2.at[c % 2],
                semd,
            )

        dep_start(0).start()

        for c in range(CHUNKS_PER_W):
            t = c * NW + wid
            b = t // CHUNKS_PER_B
            rc = t - b * CHUNKS_PER_B
            base0 = jnp.maximum(S0 * rc - 1, 0)
            base1 = jnp.maximum(2 * rc - 1, 0)
            cb = c % 2

            if c == 0:
                @plsc.parallel_loop(0, SLAB0 // L, unroll=8)
                def _(i):
                    slab0[pl.ds(i * L, L)] = zeros16

                @plsc.parallel_loop(0, SLAB1 // L, unroll=8)
                def _(i):
                    slab1[pl.ds(i * L, L)] = zeros16

            dep_start(c).wait()
            if c + 1 < CHUNKS_PER_W:
                dep_start(c + 1).start()

            @plsc.parallel_loop(0, ROWS_PER_CHUNK, unroll=2)
            def row_body(r):
                y = rc * ROWS_PER_CHUNK + r
                yv = jnp.broadcast_to(y, (L,)).astype(jnp.float32)
                y0a = lax.div(2 * y + (S0 + 1), 2 * S0) - 1
                y0av = jnp.broadcast_to(y0a, (L,)).astype(jnp.float32)
                fya = yv * (1.0 / S0) + (0.5 / S0 - 0.5) - y0av
                ly0a = jnp.clip(y0a, 0, H0 - 1) - base0
                ly1a = jnp.clip(y0a + 1, 0, H0 - 1) - base0
                ha = jnp.broadcast_to(ly0a * H0, (L,))
                hb = jnp.broadcast_to(ly1a * H0, (L,))
                w00 = (1.0 - fya) * wx0a
                w01 = (1.0 - fya) * wx1a
                w10 = fya * wx0a
                w11 = fya * wx1a
                y0b = lax.div(2 * y + (S1 + 1), 2 * S1) - 1
                y0bv = jnp.broadcast_to(y0b, (L,)).astype(jnp.float32)
                fyb = yv * (1.0 / S1) + (0.5 / S1 - 0.5) - y0bv
                ly0b = jnp.clip(y0b, 0, H1 - 1) - base1
                ly1b = jnp.clip(y0b + 1, 0, H1 - 1) - base1
                hc = jnp.broadcast_to(ly0b * H1, (L,))
                hd = jnp.broadcast_to(ly1b * H1, (L,))
                v00 = (1.0 - fyb) * wx0b
                v01 = (1.0 - fyb) * wx1b
                v10 = fyb * wx0b
                v11 = fyb * wx1b

                @plsc.parallel_loop(0, VPR, unroll=4)
                def _(v):
                    d = dep2[cb, pl.ds((r * VPR + v) * L, L)]
                    q = d * INV_STEP
                    b0 = q.astype(jnp.int32)
                    b1 = jnp.where(b0.astype(jnp.float32) * STEP > d, b0 - 1, b0)
                    b2 = jnp.where(
                        (b1.astype(jnp.float32) + 1.0) * STEP <= d, b1 + 1, b1
                    )
                    bin_ = jnp.clip(b2, 0, D - 1)

                    ta = bin_ * K0
                    xsa = jnp.broadcast_to((L // S0) * v, (L,))
                    x0 = jnp.maximum(xi0a + xsa, 0) + ta
                    x1 = jnp.minimum(xi1a + xsa, H0 - 1) + ta
                    plsc.addupdate_scatter(slab0, [ha + x0], w00)
                    plsc.addupdate_scatter(slab0, [ha + x1], w01)
                    plsc.addupdate_scatter(slab0, [hb + x0], w10)
                    plsc.addupdate_scatter(slab0, [hb + x1], w11)

                    tb = bin_ * K1
                    xsb = jnp.broadcast_to((L // S1) * v, (L,))
                    xb0 = jnp.maximum(xi0b + xsb, 0) + tb
                    xb1 = jnp.minimum(xi1b + xsb, H1 - 1) + tb
                    plsc.addupdate_scatter(slab1, [hc + xb0], v00)
                    plsc.addupdate_scatter(slab1, [hc + xb1], v01)
                    plsc.addupdate_scatter(slab1, [hd + xb0], v10)
                    plsc.addupdate_scatter(slab1, [hd + xb1], v11)

            h0 = pltpu.make_async_copy(slab0, out0_hbm.at[t], sem0)
            h1 = pltpu.make_async_copy(slab1, out1_hbm.at[t], sem1)
            h0.start()
            h1.start()
            h0.wait()
            if c + 1 < CHUNKS_PER_W:
                @plsc.parallel_loop(0, SLAB0 // L, unroll=8)
                def _(i):
                    slab0[pl.ds(i * L, L)] = zeros16

            h1.wait()
            if c + 1 < CHUNKS_PER_W:
                @plsc.parallel_loop(0, SLAB1 // L, unroll=8)
                def _(i):
                    slab1[pl.ds(i * L, L)] = zeros16

    return sc_kernel(depths_flat)


def _tc_stage(slab0, slab1, f0t, f1t):
    C0, C1 = f0t.shape[-1], f1t.shape[-1]
    K = max(C0, C1)

    def body(slab0_ref, slab1_ref, f0_ref, f1_ref, out_ref, acc0, acc1, accc):
        t = pl.program_id(0)

        @pl.when(t == 0)
        def _():
            acc0[...] = jnp.zeros_like(acc0)
            acc1[...] = jnp.zeros_like(acc1)
            accc[...] = jnp.zeros_like(accc)

        for sub in range(TC_SUB):
            tt = TC_SUB * t + sub
            b = tt // CHUNKS_PER_B
            rc = tt - b * CHUNKS_PER_B
            base0 = jnp.maximum(S0 * rc - 1, 0)
            base1 = jnp.maximum(2 * rc - 1, 0)

            s0 = slab0_ref[sub]
            win0 = f0_ref[b, pl.ds(base0, WIN0)].reshape(K0, C0)
            win1 = f1_ref[b, pl.ds(base1, WIN1)].reshape(K1, C1)
            acc0[...] += jnp.dot(
                s0, win0,
                preferred_element_type=jnp.float32,
                precision=lax.Precision.HIGHEST,
            )
            acc1[...] += jnp.dot(
                slab1_ref[sub], win1,
                preferred_element_type=jnp.float32,
                precision=lax.Precision.HIGHEST,
            )
            accc[...] += jnp.sum(s0, axis=1, keepdims=True)

        @pl.when(t == NTASK // TC_SUB - 1)
        def _():
            counts = accc[...]
            denom = jnp.maximum(counts, 1.0)
            scale = jnp.where(counts > 0.0, 1.0 / denom, 0.0)
            out_ref[0, : C0, :] = (acc0[...] * scale).T
            out_ref[0, C0:, :] = jnp.zeros((K - C0, D), jnp.float32)
            out_ref[1, :, :] = (acc1[...] * scale).T

    return pl.pallas_call(
        body,
        grid=(NTASK // TC_SUB,),
        in_specs=[
            pl.BlockSpec((TC_SUB, D, K0), lambda t: (t, 0, 0)),
            pl.BlockSpec((TC_SUB, D, K1), lambda t: (t, 0, 0)),
            pl.BlockSpec((B, H0, H0, C0), lambda t: (0, 0, 0, 0)),
            pl.BlockSpec((B, H1, H1, C1), lambda t: (0, 0, 0, 0)),
        ],
        out_specs=pl.BlockSpec((2, K, D), lambda t: (0, 0, 0)),
        out_shape=jax.ShapeDtypeStruct((2, K, D), jnp.float32),
        scratch_shapes=[
            pltpu.VMEM((D, C0), jnp.float32),
            pltpu.VMEM((D, C1), jnp.float32),
            pltpu.VMEM((D, 1), jnp.float32),
        ],
    )(slab0, slab1, f0t, f1t)


def kernel(imgs, depths, fmap0, fmap1):
    del imgs
    f0t = jnp.transpose(fmap0, (0, 2, 3, 1))
    f1t = jnp.transpose(fmap1, (0, 2, 3, 1))
    depths_flat = depths.reshape(-1)
    slab0, slab1 = _sc_stage(depths_flat)
    slab0 = slab0.reshape(NTASK, D, K0)
    slab1 = slab1.reshape(NTASK, D, K1)
    return _tc_stage(slab0, slab1, f0t, f1t)


# speedup vs baseline: 1.2897x; 1.0261x over previous
import functools
import numpy as np
import jax
import jax.numpy as jnp
from jax import lax
from jax.experimental import pallas as pl
from jax.experimental.pallas import tpu as pltpu
from jax.experimental.pallas import tpu_sc as plsc

D = 128
B = 4
H = 384
NC, NS, L = 2, 16, 16
NW = NC * NS
ROWS_PER_CHUNK = 16
CHUNKS_PER_B = H // ROWS_PER_CHUNK
NTASK = B * CHUNKS_PER_B
CHUNKS_PER_W = NTASK // NW
VPR = H // L

H0, S0, WIN0 = 96, 4, 6
H1, S1, WIN1 = 48, 8, 4
K0 = WIN0 * H0
K1 = WIN1 * H1
SLAB0 = D * K0
SLAB1 = D * K1
DEPW = ROWS_PER_CHUNK * H

STEP = 7.8125
INV_STEP = 0.128
TC_SUB = 8


def _sc_stage(depths_flat):
    mesh = plsc.VectorSubcoreMesh(
        core_axis_name="c", subcore_axis_name="s", num_cores=NC, num_subcores=NS
    )

    out_type = (
        jax.ShapeDtypeStruct((NTASK, SLAB0), jnp.float32),
        jax.ShapeDtypeStruct((NTASK, SLAB1), jnp.float32),
    )

    @functools.partial(
        pl.kernel,
        out_type=out_type,
        mesh=mesh,
        compiler_params=pltpu.CompilerParams(needs_layout_passes=False),
        scratch_types=[
            pltpu.VMEM((SLAB0,), jnp.float32),
            pltpu.VMEM((SLAB1,), jnp.float32),
            pltpu.VMEM((2, DEPW), jnp.float32),
            pltpu.SemaphoreType.DMA,
            pltpu.SemaphoreType.DMA,
            pltpu.SemaphoreType.DMA,
        ],
    )
    def sc_kernel(dep_hbm, out0_hbm, out1_hbm, slab0, slab1, dep2, sem0, sem1, semd):
        wid = lax.axis_index("s") * NC + lax.axis_index("c")

        zeros16 = jnp.zeros((L,), jnp.float32)
        lane = lax.iota(jnp.int32, L)
        lanef = lane.astype(jnp.float32)
        xi0a = lax.div(2 * lane + (S0 + 1), jnp.full((L,), 2 * S0, jnp.int32)) - 1
        fxa = lanef * (1.0 / S0) + (0.5 / S0 - 0.5) - xi0a.astype(jnp.float32)
        xi1a = xi0a + 1
        wx0a = 1.0 - fxa
        wx1a = fxa
        xi0b = lax.div(2 * lane + (S1 + 1), jnp.full((L,), 2 * S1, jnp.int32)) - 1
        fxb = lanef * (1.0 / S1) + (0.5 / S1 - 0.5) - xi0b.astype(jnp.float32)
        xi1b = xi0b + 1
        wx0b = 1.0 - fxb
        wx1b = fxb

        def dep_start(c):
            t = c * NW + wid
            b = t // CHUNKS_PER_B
            rc = t - b * CHUNKS_PER_B
            return pltpu.make_async_copy(
                dep_hbm.at[pl.ds(b * (H * H) + rc * DEPW, DEPW)],
                dep2.at[c % 2],
                semd,
            )

        dep_start(0).start()

        for c in range(CHUNKS_PER_W):
            t = c * NW + wid
            b = t // CHUNKS_PER_B
            rc = t - b * CHUNKS_PER_B
            base0 = jnp.maximum(S0 * rc - 1, 0)
            base1 = jnp.maximum(2 * rc - 1, 0)
            cb = c % 2

            if c == 0:
                @plsc.parallel_loop(0, SLAB0 // L, unroll=8)
                def _(i):
                    slab0[pl.ds(i * L, L)] = zeros16

                @plsc.parallel_loop(0, SLAB1 // L, unroll=8)
                def _(i):
                    slab1[pl.ds(i * L, L)] = zeros16

            dep_start(c).wait()
            if c + 1 < CHUNKS_PER_W:
                dep_start(c + 1).start()

            @plsc.parallel_loop(0, ROWS_PER_CHUNK, unroll=2)
            def row_body(r):
                y = rc * ROWS_PER_CHUNK + r
                yv = jnp.broadcast_to(y, (L,)).astype(jnp.float32)
                y0a = lax.div(2 * y + (S0 + 1), 2 * S0) - 1
                y0av = jnp.broadcast_to(y0a, (L,)).astype(jnp.float32)
                fya = yv * (1.0 / S0) + (0.5 / S0 - 0.5) - y0av
                ly0a = jnp.clip(y0a, 0, H0 - 1) - base0
                ly1a = jnp.clip(y0a + 1, 0, H0 - 1) - base0
                ha = jnp.broadcast_to(ly0a * H0, (L,))
                hb = jnp.broadcast_to(ly1a * H0, (L,))
                w00 = (1.0 - fya) * wx0a
                w01 = (1.0 - fya) * wx1a
                w10 = fya * wx0a
                w11 = fya * wx1a
                y0b = lax.div(2 * y + (S1 + 1), 2 * S1) - 1
                y0bv = jnp.broadcast_to(y0b, (L,)).astype(jnp.float32)
                fyb = yv * (1.0 / S1) + (0.5 / S1 - 0.5) - y0bv
                ly0b = jnp.clip(y0b, 0, H1 - 1) - base1
                ly1b = jnp.clip(y0b + 1, 0, H1 - 1) - base1
                hc = jnp.broadcast_to(ly0b * H1, (L,))
                hd = jnp.broadcast_to(ly1b * H1, (L,))
                v00 = (1.0 - fyb) * wx0b
                v01 = (1.0 - fyb) * wx1b
                v10 = fyb * wx0b
                v11 = fyb * wx1b

                @plsc.parallel_loop(0, VPR, unroll=4)
                def _(v):
                    d = dep2[cb, pl.ds((r * VPR + v) * L, L)]
                    q = d * INV_STEP
                    b0 = q.astype(jnp.int32)
                    b1 = jnp.where(b0.astype(jnp.float32) * STEP > d, b0 - 1, b0)
                    b2 = jnp.where(
                        (b1.astype(jnp.float32) + 1.0) * STEP <= d, b1 + 1, b1
                    )
                    bin_ = jnp.clip(b2, 0, D - 1)

                    ta = bin_ * K0
                    xsa = jnp.broadcast_to((L // S0) * v, (L,))
                    x0 = jnp.maximum(xi0a + xsa, 0) + ta
                    x1 = jnp.minimum(xi1a + xsa, H0 - 1) + ta
                    plsc.addupdate_scatter(slab0, [ha + x0], w00)
                    plsc.addupdate_scatter(slab0, [ha + x1], w01)
                    plsc.addupdate_scatter(slab0, [hb + x0], w10)
                    plsc.addupdate_scatter(slab0, [hb + x1], w11)

                    tb = bin_ * K1
                    xsb = jnp.broadcast_to((L // S1) * v, (L,))
                    xb0 = jnp.maximum(xi0b + xsb, 0) + tb
                    xb1 = jnp.minimum(xi1b + xsb, H1 - 1) + tb
                    plsc.addupdate_scatter(slab1, [hc + xb0], v00)
                    plsc.addupdate_scatter(slab1, [hc + xb1], v01)
                    plsc.addupdate_scatter(slab1, [hd + xb0], v10)
                    plsc.addupdate_scatter(slab1, [hd + xb1], v11)

            h0 = pltpu.make_async_copy(slab0, out0_hbm.at[t], sem0)
            h1 = pltpu.make_async_copy(slab1, out1_hbm.at[t], sem1)
            h0.start()
            h1.start()
            h0.wait()
            if c + 1 < CHUNKS_PER_W:
                @plsc.parallel_loop(0, SLAB0 // L, unroll=8)
                def _(i):
                    slab0[pl.ds(i * L, L)] = zeros16

            h1.wait()
            if c + 1 < CHUNKS_PER_W:
                @plsc.parallel_loop(0, SLAB1 // L, unroll=8)
                def _(i):
                    slab1[pl.ds(i * L, L)] = zeros16

    return sc_kernel(depths_flat)


def _tc_stage(slab0, slab1, f0t, f1t):
    C0, C1 = f0t.shape[-1], f1t.shape[-1]
    K = max(C0, C1)

    def body(slab0_ref, slab1_ref, f0_ref, f1_ref, out_ref, acc0, acc1, accc):
        t = pl.program_id(0)

        @pl.when(t == 0)
        def _():
            acc0[...] = jnp.zeros_like(acc0)
            acc1[...] = jnp.zeros_like(acc1)
            accc[...] = jnp.zeros_like(accc)

        for sub in range(TC_SUB):
            tt = TC_SUB * t + sub
            b = tt // CHUNKS_PER_B
            rc = tt - b * CHUNKS_PER_B
            base0 = jnp.maximum(S0 * rc - 1, 0)
            base1 = jnp.maximum(2 * rc - 1, 0)

            s0 = slab0_ref[sub]
            win0 = f0_ref[0, pl.ds(base0, WIN0)].reshape(K0, C0)
            win1 = f1_ref[0, pl.ds(base1, WIN1)].reshape(K1, C1)
            acc0[...] += jnp.dot(
                s0, win0,
                preferred_element_type=jnp.float32,
                precision=lax.Precision.HIGHEST,
            )
            acc1[...] += jnp.dot(
                slab1_ref[sub], win1,
                preferred_element_type=jnp.float32,
                precision=lax.Precision.HIGHEST,
            )
            accc[...] += jnp.sum(s0, axis=1, keepdims=True)

        @pl.when(t == NTASK // TC_SUB - 1)
        def _():
            counts = accc[...]
            denom = jnp.maximum(counts, 1.0)
            scale = jnp.where(counts > 0.0, 1.0 / denom, 0.0)
            out_ref[0, : C0, :] = (acc0[...] * scale).T
            out_ref[0, C0:, :] = jnp.zeros((K - C0, D), jnp.float32)
            out_ref[1, :, :] = (acc1[...] * scale).T

    return pl.pallas_call(
        body,
        grid=(NTASK // TC_SUB,),
        in_specs=[
            pl.BlockSpec((TC_SUB, D, K0), lambda t: (t, 0, 0)),
            pl.BlockSpec((TC_SUB, D, K1), lambda t: (t, 0, 0)),
            pl.BlockSpec(
                (1, H0, H0, C0),
                lambda t: ((TC_SUB * t) // CHUNKS_PER_B, 0, 0, 0),
            ),
            pl.BlockSpec(
                (1, H1, H1, C1),
                lambda t: ((TC_SUB * t) // CHUNKS_PER_B, 0, 0, 0),
            ),
        ],
        out_specs=pl.BlockSpec((2, K, D), lambda t: (0, 0, 0)),
        out_shape=jax.ShapeDtypeStruct((2, K, D), jnp.float32),
        scratch_shapes=[
            pltpu.VMEM((D, C0), jnp.float32),
            pltpu.VMEM((D, C1), jnp.float32),
            pltpu.VMEM((D, 1), jnp.float32),
        ],
    )(slab0, slab1, f0t, f1t)


def kernel(imgs, depths, fmap0, fmap1):
    del imgs
    f0t = jnp.transpose(fmap0, (0, 2, 3, 1))
    f1t = jnp.transpose(fmap1, (0, 2, 3, 1))
    depths_flat = depths.reshape(-1)
    slab0, slab1 = _sc_stage(depths_flat)
    slab0 = slab0.reshape(NTASK, D, K0)
    slab1 = slab1.reshape(NTASK, D, K1)
    return _tc_stage(slab0, slab1, f0t, f1t)


# speedup vs baseline: 1.2897x; 1.0000x over previous
import functools
import jax
import jax.numpy as jnp
from jax import lax
from jax.experimental import pallas as pl
from jax.experimental.pallas import tpu as pltpu
from jax.experimental.pallas import tpu_sc as plsc

D = 128
B = 4
H = 384
NC, NS, L = 2, 16, 16
NW = NC * NS
ROWS_PER_CHUNK = 16
CHUNKS_PER_B = H // ROWS_PER_CHUNK
NTASK = B * CHUNKS_PER_B
CHUNKS_PER_W = NTASK // NW
VPR = H // L

H0, S0, WIN0 = 96, 4, 6
H1, S1, WIN1 = 48, 8, 4
K0 = WIN0 * H0
K1 = WIN1 * H1
SLAB0 = D * K0
SLAB1 = D * K1
DEPW = ROWS_PER_CHUNK * H

STEP = 7.8125
INV_STEP = 0.128
TC_SUB = 8


def _sc_stage(depths_flat):
    mesh = plsc.VectorSubcoreMesh(
        core_axis_name="c", subcore_axis_name="s", num_cores=NC, num_subcores=NS
    )

    out_type = (
        jax.ShapeDtypeStruct((NTASK, SLAB0), jnp.float32),
        jax.ShapeDtypeStruct((NTASK, SLAB1), jnp.float32),
    )

    @functools.partial(
        pl.kernel,
        out_type=out_type,
        mesh=mesh,
        compiler_params=pltpu.CompilerParams(needs_layout_passes=False),
        scratch_types=[
            pltpu.VMEM((SLAB0,), jnp.float32),
            pltpu.VMEM((SLAB1,), jnp.float32),
            pltpu.VMEM((2, DEPW), jnp.float32),
            pltpu.SemaphoreType.DMA,
            pltpu.SemaphoreType.DMA,
            pltpu.SemaphoreType.DMA,
        ],
    )
    def sc_kernel(dep_hbm, out0_hbm, out1_hbm, slab0, slab1, dep2, sem0, sem1, semd):
        wid = lax.axis_index("s") * NC + lax.axis_index("c")

        zeros16 = jnp.zeros((L,), jnp.float32)
        lane = lax.iota(jnp.int32, L)
        lanef = lane.astype(jnp.float32)
        xi0a = lax.div(2 * lane + (S0 + 1), jnp.full((L,), 2 * S0, jnp.int32)) - 1
        fxa = lanef * (1.0 / S0) + (0.5 / S0 - 0.5) - xi0a.astype(jnp.float32)
        xi1a = xi0a + 1
        wx0a = 1.0 - fxa
        wx1a = fxa
        xi0b = lax.div(2 * lane + (S1 + 1), jnp.full((L,), 2 * S1, jnp.int32)) - 1
        fxb = lanef * (1.0 / S1) + (0.5 / S1 - 0.5) - xi0b.astype(jnp.float32)
        xi1b = xi0b + 1
        wx0b = 1.0 - fxb
        wx1b = fxb

        def dep_start(c):
            t = c * NW + wid
            b = t // CHUNKS_PER_B
            rc = t - b * CHUNKS_PER_B
            return pltpu.make_async_copy(
                dep_hbm.at[pl.ds(b * (H * H) + rc * DEPW, DEPW)],
                dep2.at[c % 2],
                semd,
            )

        dep_start(0).start()

        for c in range(CHUNKS_PER_W):
            t = c * NW + wid
            b = t // CHUNKS_PER_B
            rc = t - b * CHUNKS_PER_B
            base0 = jnp.maximum(S0 * rc - 1, 0)
            base1 = jnp.maximum(2 * rc - 1, 0)
            cb = c % 2

            if c == 0:
                @plsc.parallel_loop(0, SLAB0 // L, unroll=8)
                def _(i):
                    slab0[pl.ds(i * L, L)] = zeros16

                @plsc.parallel_loop(0, SLAB1 // L, unroll=8)
                def _(i):
                    slab1[pl.ds(i * L, L)] = zeros16

            dep_start(c).wait()
            if c + 1 < CHUNKS_PER_W:
                dep_start(c + 1).start()

            @plsc.parallel_loop(0, ROWS_PER_CHUNK, unroll=2)
            def row_body(r):
                y = rc * ROWS_PER_CHUNK + r
                yv = jnp.broadcast_to(y, (L,)).astype(jnp.float32)
                y0a = lax.div(2 * y + (S0 + 1), 2 * S0) - 1
                y0av = jnp.broadcast_to(y0a, (L,)).astype(jnp.float32)
                fya = yv * (1.0 / S0) + (0.5 / S0 - 0.5) - y0av
                ly0a = jnp.clip(y0a, 0, H0 - 1) - base0
                ly1a = jnp.clip(y0a + 1, 0, H0 - 1) - base0
                ha = jnp.broadcast_to(ly0a * H0, (L,))
                hb = jnp.broadcast_to(ly1a * H0, (L,))
                w00 = (1.0 - fya) * wx0a
                w01 = (1.0 - fya) * wx1a
                w10 = fya * wx0a
                w11 = fya * wx1a
                y0b = lax.div(2 * y + (S1 + 1), 2 * S1) - 1
                y0bv = jnp.broadcast_to(y0b, (L,)).astype(jnp.float32)
                fyb = yv * (1.0 / S1) + (0.5 / S1 - 0.5) - y0bv
                ly0b = jnp.clip(y0b, 0, H1 - 1) - base1
                ly1b = jnp.clip(y0b + 1, 0, H1 - 1) - base1
                hc = jnp.broadcast_to(ly0b * H1, (L,))
                hd = jnp.broadcast_to(ly1b * H1, (L,))
                v00 = (1.0 - fyb) * wx0b
                v01 = (1.0 - fyb) * wx1b
                v10 = fyb * wx0b
                v11 = fyb * wx1b

                @plsc.parallel_loop(0, VPR, unroll=4)
                def _(v):
                    d = dep2[cb, pl.ds((r * VPR + v) * L, L)]
                    q = d * INV_STEP
                    b0 = q.astype(jnp.int32)
                    b1 = jnp.where(b0.astype(jnp.float32) * STEP > d, b0 - 1, b0)
                    b2 = jnp.where(
                        (b1.astype(jnp.float32) + 1.0) * STEP <= d, b1 + 1, b1
                    )
                    bin_ = jnp.clip(b2, 0, D - 1)

                    ta = bin_ * K0
                    xsa = jnp.broadcast_to((L // S0) * v, (L,))
                    x0 = jnp.maximum(xi0a + xsa, 0) + ta
                    x1 = jnp.minimum(xi1a + xsa, H0 - 1) + ta
                    plsc.addupdate_scatter(slab0, [ha + x0], w00)
                    plsc.addupdate_scatter(slab0, [ha + x1], w01)
                    plsc.addupdate_scatter(slab0, [hb + x0], w10)
                    plsc.addupdate_scatter(slab0, [hb + x1], w11)

                    tb = bin_ * K1
                    xsb = jnp.broadcast_to((L // S1) * v, (L,))
                    xb0 = jnp.maximum(xi0b + xsb, 0) + tb
                    xb1 = jnp.minimum(xi1b + xsb, H1 - 1) + tb
                    plsc.addupdate_scatter(slab1, [hc + xb0], v00)
                    plsc.addupdate_scatter(slab1, [hc + xb1], v01)
                    plsc.addupdate_scatter(slab1, [hd + xb0], v10)
                    plsc.addupdate_scatter(slab1, [hd + xb1], v11)

            h0 = pltpu.make_async_copy(slab0, out0_hbm.at[t], sem0)
            h1 = pltpu.make_async_copy(slab1, out1_hbm.at[t], sem1)
            h0.start()
            h1.start()
            h0.wait()
            if c + 1 < CHUNKS_PER_W:
                @plsc.parallel_loop(0, SLAB0 // L, unroll=8)
                def _(i):
                    slab0[pl.ds(i * L, L)] = zeros16

            h1.wait()
            if c + 1 < CHUNKS_PER_W:
                @plsc.parallel_loop(0, SLAB1 // L, unroll=8)
                def _(i):
                    slab1[pl.ds(i * L, L)] = zeros16

    return sc_kernel(depths_flat)


def _tc_stage(slab0, slab1, f0t, f1t):
    C0, C1 = f0t.shape[-1], f1t.shape[-1]
    K = max(C0, C1)

    def body(slab0_ref, slab1_ref, f0_ref, f1_ref, out_ref, acc0, acc1, accc):
        t = pl.program_id(0)

        @pl.when(t == 0)
        def _():
            acc0[...] = jnp.zeros_like(acc0)
            acc1[...] = jnp.zeros_like(acc1)
            accc[...] = jnp.zeros_like(accc)

        for sub in range(TC_SUB):
            tt = TC_SUB * t + sub
            b = tt // CHUNKS_PER_B
            rc = tt - b * CHUNKS_PER_B
            base0 = jnp.maximum(S0 * rc - 1, 0)
            base1 = jnp.maximum(2 * rc - 1, 0)

            s0 = slab0_ref[sub]
            win0 = f0_ref[0, pl.ds(base0, WIN0)].reshape(K0, C0)
            win1 = f1_ref[0, pl.ds(base1, WIN1)].reshape(K1, C1)
            acc0[...] += jnp.dot(
                s0, win0,
                preferred_element_type=jnp.float32,
                precision=lax.Precision.HIGHEST,
            )
            acc1[...] += jnp.dot(
                slab1_ref[sub], win1,
                preferred_element_type=jnp.float32,
                precision=lax.Precision.HIGHEST,
            )
            accc[...] += jnp.sum(s0, axis=1, keepdims=True)

        @pl.when(t == NTASK // TC_SUB - 1)
        def _():
            counts = accc[...]
            denom = jnp.maximum(counts, 1.0)
            scale = jnp.where(counts > 0.0, 1.0 / denom, 0.0)
            out_ref[0, : C0, :] = (acc0[...] * scale).T
            out_ref[0, C0:, :] = jnp.zeros((K - C0, D), jnp.float32)
            out_ref[1, :, :] = (acc1[...] * scale).T

    return pl.pallas_call(
        body,
        grid=(NTASK // TC_SUB,),
        in_specs=[
            pl.BlockSpec((TC_SUB, D, K0), lambda t: (t, 0, 0)),
            pl.BlockSpec((TC_SUB, D, K1), lambda t: (t, 0, 0)),
            pl.BlockSpec(
                (1, H0, H0, C0),
                lambda t: ((TC_SUB * t) // CHUNKS_PER_B, 0, 0, 0),
            ),
            pl.BlockSpec(
                (1, H1, H1, C1),
                lambda t: ((TC_SUB * t) // CHUNKS_PER_B, 0, 0, 0),
            ),
        ],
        out_specs=pl.BlockSpec((2, K, D), lambda t: (0, 0, 0)),
        out_shape=jax.ShapeDtypeStruct((2, K, D), jnp.float32),
        scratch_shapes=[
            pltpu.VMEM((D, C0), jnp.float32),
            pltpu.VMEM((D, C1), jnp.float32),
            pltpu.VMEM((D, 1), jnp.float32),
        ],
    )(slab0, slab1, f0t, f1t)


def kernel(imgs, depths, fmap0, fmap1):
    del imgs
    f0t = jnp.transpose(fmap0, (0, 2, 3, 1))
    f1t = jnp.transpose(fmap1, (0, 2, 3, 1))
    depths_flat = depths.reshape(-1)
    slab0, slab1 = _sc_stage(depths_flat)
    slab0 = slab0.reshape(NTASK, D, K0)
    slab1 = slab1.reshape(NTASK, D, K1)
    return _tc_stage(slab0, slab1, f0t, f1t)
